# baseline (speedup 1.0000x reference)
import functools

import jax
import jax.numpy as jnp
from jax import lax
from jax.experimental import pallas as pl
from jax.experimental.pallas import tpu as pltpu
from jax.experimental.pallas import tpu_sc as plsc

VEC = 64
VECP = 128
BATCH = 4096
HIST = 200
VOCAB1 = 1000001
VROWS = 1000064
NC, NS = 2, 16
NW = NC * NS
BPW = BATCH // NW
H0 = 128
H1 = HIST - H0
TMAIN = 7812
TAIL0 = TMAIN * VECP
L = 16


def _transpose_body(wt_hbm, wtail_hbm, wp_hbm,
                    in0, in1, out0, out1, isem0, isem1, osem0, osem1):
    wid = lax.axis_index("s") * NC + lax.axis_index("c")
    n = jnp.where(wid < 4, 245, 244)
    c0 = jnp.where(wid < 4, wid * 245, 4 * 245 + (wid - 4) * 244)

    inb = (in0, in1)
    outb = (out0, out1)
    isem = (isem0, isem1)
    osem = (osem0, osem1)

    def fire_in(j, p):
        pltpu.async_copy(
            wt_hbm.at[:, pl.ds((c0 + j) * VECP, VECP)], inb[p], isem[p])

    def wait_in(p):
        pltpu.make_async_copy(
            wt_hbm.at[:, pl.ds(0, VECP)], inb[p], isem[p]).wait()

    def fire_out(j, p):
        pltpu.async_copy(
            outb[p], wp_hbm.at[pl.ds((c0 + j) * VECP, VECP)], osem[p])

    def wait_out(p):
        pltpu.make_async_copy(
            outb[p], wp_hbm.at[pl.ds(0, VECP)], osem[p]).wait()

    rowv = [lax.iota(jnp.int32, L) + L * k for k in range(VECP // L)]

    def transpose(p):
        src, dst = inb[p], outb[p]
        for d in range(VEC):
            colv = jnp.full((L,), d, jnp.int32)
            for k in range(VECP // L):
                v = src[d, pl.ds(L * k, L)]
                plsc.store_scatter(dst, [rowv[k], colv], v)

    fire_in(0, 0)

    @pl.loop(0, 246, step=2)
    def _steps(g):
        for p in (0, 1):
            j = g + p

            @pl.when(j < n)
            def _():
                wait_in(p)

                @pl.when(j + 1 < n)
                def _():
                    fire_in(j + 1, 1 - p)

                @pl.when(j >= 2)
                def _():
                    wait_out(p)
                transpose(p)
                fire_out(j, p)

    wait_out(0)
    wait_out(1)

    @pl.when(wid == NW - 1)
    def _():
        pltpu.sync_copy(wtail_hbm, out0)
        pltpu.sync_copy(out0, wp_hbm.at[pl.ds(TAIL0, VECP)])


_transpose = functools.partial(
    pl.kernel,
    out_type=jax.ShapeDtypeStruct((VROWS, VECP), jnp.float32),
    mesh=plsc.VectorSubcoreMesh(core_axis_name="c", subcore_axis_name="s"),
    scratch_types=[
        pltpu.VMEM((VEC, VECP), jnp.float32),
        pltpu.VMEM((VEC, VECP), jnp.float32),
        pltpu.VMEM((VECP, VECP), jnp.float32),
        pltpu.VMEM((VECP, VECP), jnp.float32),
        pltpu.SemaphoreType.DMA,
        pltpu.SemaphoreType.DMA,
        pltpu.SemaphoreType.DMA,
        pltpu.SemaphoreType.DMA,
    ],
    compiler_params=pltpu.CompilerParams(
        use_tc_tiling_on_sc=True, needs_layout_passes=False),
)(_transpose_body)


def _emb_body(inds_hbm, w_hbm, out_hbm,
              idx_v, rows0, rows1, gsem0, gsem1, wsem0, wsem1):
    wid = lax.axis_index("s") * NC + lax.axis_index("c")
    b0 = wid * BPW

    rows = (rows0, rows1)
    gsem = (gsem0, gsem1)
    wsem = (wsem0, wsem1)

    pltpu.sync_copy(inds_hbm.at[pl.ds(b0, BPW)], idx_v)

    def fire_gathers(bl, p):
        pltpu.async_copy(
            w_hbm.at[idx_v.at[bl, pl.ds(0, H0)]],
            rows[p].at[pl.ds(0, H0)], gsem[p])
        pltpu.async_copy(
            w_hbm.at[idx_v.at[bl, pl.ds(H0, H1)]],
            rows[p].at[pl.ds(H0, H1)], gsem[p])

    def wait_gathers(p):
        pltpu.make_async_copy(w_hbm.at[pl.ds(0, HIST)], rows[p], gsem[p]).wait()

    def writeback(bl, p):
        pltpu.async_copy(
            rows[p], out_hbm.at[pl.ds((b0 + bl) * HIST, HIST)], wsem[p])

    def wait_writeback(bl, p):
        pltpu.make_async_copy(
            rows[p], out_hbm.at[pl.ds((b0 + bl) * HIST, HIST)], wsem[p]).wait()

    fire_gathers(0, 0)

    @pl.loop(0, BPW, step=2)
    def _steps(g):
        for p in (0, 1):
            bl = g + p
            np_ = 1 - p

            @pl.when(bl + 1 < BPW)
            def _():
                @pl.when(bl >= 1)
                def _():
                    wait_writeback(bl - 1, np_)
                fire_gathers(bl + 1, np_)

            wait_gathers(p)
            writeback(bl, p)

    wait_writeback(BPW - 2, 0)
    wait_writeback(BPW - 1, 1)


_emb = functools.partial(
    pl.kernel,
    out_type=jax.ShapeDtypeStruct((BATCH * HIST, VECP), jnp.float32),
    mesh=plsc.VectorSubcoreMesh(core_axis_name="c", subcore_axis_name="s"),
    scratch_types=[
        pltpu.VMEM((BPW, HIST), jnp.int32),
        pltpu.VMEM((HIST, VECP), jnp.float32),
        pltpu.VMEM((HIST, VECP), jnp.float32),
        pltpu.SemaphoreType.DMA,
        pltpu.SemaphoreType.DMA,
        pltpu.SemaphoreType.DMA,
        pltpu.SemaphoreType.DMA,
    ],
    compiler_params=pltpu.CompilerParams(use_tc_tiling_on_sc=True),
)(_emb_body)


@jax.jit
def kernel(inds, W):
    Wtail = jnp.pad(W[TAIL0:], ((0, VECP - (VOCAB1 - TAIL0)), (0, VECP - VEC)))
    Wp = _transpose(W.T, Wtail)
    out = _emb(inds, Wp)
    return out[:, :VEC].reshape(BATCH, HIST, VEC)


# speedup vs baseline: 1.0116x; 1.0116x over previous
import functools

import jax
import jax.numpy as jnp
from jax import lax
from jax.experimental import pallas as pl
from jax.experimental.pallas import tpu as pltpu
from jax.experimental.pallas import tpu_sc as plsc

VEC = 64
VECP = 128
BATCH = 4096
HIST = 200
VOCAB1 = 1000001
VROWS = 1000064
NC, NS = 2, 16
NW = NC * NS
BPW = BATCH // NW
H0 = 128
H1 = HIST - H0
TMAIN = 7812
TAIL0 = TMAIN * VECP
L = 16


def _transpose_body(wt_hbm, wtail_hbm, wp_hbm,
                    in0, in1, out0, out1, isem0, isem1, osem0, osem1):
    wid = lax.axis_index("s") * NC + lax.axis_index("c")
    n = jnp.where(wid < 4, 245, 244)
    c0 = jnp.where(wid < 4, wid * 245, 4 * 245 + (wid - 4) * 244)

    inb = (in0, in1)
    outb = (out0, out1)
    isem = (isem0, isem1)
    osem = (osem0, osem1)

    def fire_in(j, p):
        pltpu.async_copy(
            wt_hbm.at[:, pl.ds((c0 + j) * VECP, VECP)], inb[p], isem[p])

    def wait_in(p):
        pltpu.make_async_copy(
            wt_hbm.at[:, pl.ds(0, VECP)], inb[p], isem[p]).wait()

    BLK = VECP * VECP

    def fire_out(j, p):
        pltpu.async_copy(
            outb[p], wp_hbm.at[pl.ds((c0 + j) * BLK, BLK)], osem[p])

    def wait_out(p):
        pltpu.make_async_copy(
            outb[p], wp_hbm.at[pl.ds(0, BLK)], osem[p]).wait()

    rowv = [(lax.iota(jnp.int32, L) + L * k) * VECP for k in range(VECP // L)]

    def transpose(p):
        src, dst = inb[p], outb[p]
        for d in range(VEC):
            vs = [src[d, pl.ds(L * k, L)] for k in range(VECP // L)]
            idxs = [rv + d for rv in rowv]
            for k in range(VECP // L):
                plsc.store_scatter(dst, [idxs[k]], vs[k])

    fire_in(0, 0)

    @pl.loop(0, 246, step=2)
    def _steps(g):
        for p in (0, 1):
            j = g + p

            @pl.when(j < n)
            def _():
                wait_in(p)

                @pl.when(j + 1 < n)
                def _():
                    fire_in(j + 1, 1 - p)

                @pl.when(j >= 2)
                def _():
                    wait_out(p)
                transpose(p)
                fire_out(j, p)

    wait_out(0)
    wait_out(1)

    @pl.when(wid == NW - 1)
    def _():
        pltpu.sync_copy(wtail_hbm, out0)
        pltpu.sync_copy(out0, wp_hbm.at[pl.ds(TAIL0 * VECP, BLK)])


_transpose = functools.partial(
    pl.kernel,
    out_type=jax.ShapeDtypeStruct((VROWS * VECP,), jnp.float32),
    mesh=plsc.VectorSubcoreMesh(core_axis_name="c", subcore_axis_name="s"),
    scratch_types=[
        pltpu.VMEM((VEC, VECP), jnp.float32),
        pltpu.VMEM((VEC, VECP), jnp.float32),
        pltpu.VMEM((VECP * VECP,), jnp.float32),
        pltpu.VMEM((VECP * VECP,), jnp.float32),
        pltpu.SemaphoreType.DMA,
        pltpu.SemaphoreType.DMA,
        pltpu.SemaphoreType.DMA,
        pltpu.SemaphoreType.DMA,
    ],
    compiler_params=pltpu.CompilerParams(
        use_tc_tiling_on_sc=True, needs_layout_passes=False),
)(_transpose_body)


def _emb_body(inds_hbm, w_hbm, out_hbm,
              idx_v, rows0, rows1, gsem0, gsem1, wsem0, wsem1):
    wid = lax.axis_index("s") * NC + lax.axis_index("c")
    b0 = wid * BPW

    rows = (rows0, rows1)
    gsem = (gsem0, gsem1)
    wsem = (wsem0, wsem1)

    pltpu.sync_copy(inds_hbm.at[pl.ds(b0, BPW)], idx_v)

    def fire_gathers(bl, p):
        pltpu.async_copy(
            w_hbm.at[idx_v.at[bl, pl.ds(0, H0)]],
            rows[p].at[pl.ds(0, H0)], gsem[p])
        pltpu.async_copy(
            w_hbm.at[idx_v.at[bl, pl.ds(H0, H1)]],
            rows[p].at[pl.ds(H0, H1)], gsem[p])

    def wait_gathers(p):
        pltpu.make_async_copy(w_hbm.at[pl.ds(0, HIST)], rows[p], gsem[p]).wait()

    def writeback(bl, p):
        pltpu.async_copy(
            rows[p], out_hbm.at[pl.ds((b0 + bl) * HIST, HIST)], wsem[p])

    def wait_writeback(bl, p):
        pltpu.make_async_copy(
            rows[p], out_hbm.at[pl.ds((b0 + bl) * HIST, HIST)], wsem[p]).wait()

    fire_gathers(0, 0)

    @pl.loop(0, BPW, step=2)
    def _steps(g):
        for p in (0, 1):
            bl = g + p
            np_ = 1 - p

            @pl.when(bl + 1 < BPW)
            def _():
                @pl.when(bl >= 1)
                def _():
                    wait_writeback(bl - 1, np_)
                fire_gathers(bl + 1, np_)

            wait_gathers(p)
            writeback(bl, p)

    wait_writeback(BPW - 2, 0)
    wait_writeback(BPW - 1, 1)


_emb = functools.partial(
    pl.kernel,
    out_type=jax.ShapeDtypeStruct((BATCH * HIST, VECP), jnp.float32),
    mesh=plsc.VectorSubcoreMesh(core_axis_name="c", subcore_axis_name="s"),
    scratch_types=[
        pltpu.VMEM((BPW, HIST), jnp.int32),
        pltpu.VMEM((HIST, VECP), jnp.float32),
        pltpu.VMEM((HIST, VECP), jnp.float32),
        pltpu.SemaphoreType.DMA,
        pltpu.SemaphoreType.DMA,
        pltpu.SemaphoreType.DMA,
        pltpu.SemaphoreType.DMA,
    ],
    compiler_params=pltpu.CompilerParams(use_tc_tiling_on_sc=True),
)(_emb_body)


@jax.jit
def kernel(inds, W):
    Wtail = jnp.pad(W[TAIL0:], ((0, VECP - (VOCAB1 - TAIL0)), (0, VECP - VEC)))
    Wp = _transpose(W.T, Wtail.reshape(-1)).reshape(VROWS, VECP)
    out = _emb(inds, Wp)
    return out[:, :VEC].reshape(BATCH, HIST, VEC)
